# Initial kernel scaffold; baseline (speedup 1.0000x reference)
import jax
import jax.numpy as jnp
from jax.experimental import pallas as pl


def kernel(indices, R, C_R, entity_table, W_p, W_p_c):
    raise NotImplementedError("write your pallas kernel here")


# speedup vs baseline: 1.5608x; 1.5608x over previous
import functools

import jax
import jax.numpy as jnp
from jax import lax
from jax.experimental import pallas as pl
from jax.experimental.pallas import tpu as pltpu
from jax.experimental.pallas import tpu_sc as plsc

_BATCH = 16384
_FIELDS = 26
_D = 32
_BF = _BATCH * _FIELDS

_NC = 2
_NS = 16
_NW = _NC * _NS
_PER_W = _BF // _NW
_ILEN = 128
_IDX_ROWS = _PER_W // _ILEN
_SUB = 8
_CHUNK = _SUB * _ILEN
_OUTER = _PER_W // _CHUNK


def _gather_body(table_hbm, idx_hbm, out_hbm, idx_v, rows_v, sem):
    wid = lax.axis_index("s") * _NC + lax.axis_index("c")
    pltpu.sync_copy(idx_hbm.at[pl.ds(wid * _IDX_ROWS, _IDX_ROWS)], idx_v)

    def outer(t, carry):
        cps = [
            pltpu.async_copy(
                table_hbm.at[idx_v.at[t * _SUB + b]],
                rows_v.at[pl.ds(b * _ILEN, _ILEN)],
                sem,
            )
            for b in range(_SUB)
        ]
        for cp in cps:
            cp.wait()
        row0 = wid * _PER_W + t * _CHUNK
        pltpu.sync_copy(rows_v, out_hbm.at[pl.ds(row0, _CHUNK)])
        return carry

    lax.fori_loop(0, _OUTER, outer, 0)


@functools.partial(
    pl.kernel,
    out_type=jax.ShapeDtypeStruct((_BF, _D), jnp.float32),
    mesh=plsc.VectorSubcoreMesh(core_axis_name="c", subcore_axis_name="s"),
    compiler_params=pltpu.CompilerParams(use_tc_tiling_on_sc=False),
    scratch_types=[
        pltpu.VMEM((_IDX_ROWS, _ILEN), jnp.int32),
        pltpu.VMEM((_CHUNK, _D), jnp.float32),
        pltpu.SemaphoreType.DMA,
    ],
)
def _sc_gather(table_hbm, idx_hbm, out_hbm, idx_v, rows_v, sem):
    _gather_body(table_hbm, idx_hbm, out_hbm, idx_v, rows_v, sem)


def _proj_body(r_ref, wpt_ref, cr_ref, wpct_ref, zr_ref, zc_ref):
    zr_ref[...] = jnp.dot(r_ref[...], wpt_ref[...],
                          preferred_element_type=jnp.float32)
    zc_ref[...] = jnp.dot(cr_ref[...], wpct_ref[...],
                          preferred_element_type=jnp.float32)


_tc_proj = pl.pallas_call(
    _proj_body,
    out_shape=[
        jax.ShapeDtypeStruct((500, 128), jnp.float32),
        jax.ShapeDtypeStruct((500, 128), jnp.float32),
    ],
)


def kernel(indices, R, C_R, entity_table, W_p, W_p_c):
    idx2d = indices.reshape(_BF // _ILEN, _ILEN).astype(jnp.int32)
    ent_flat = _sc_gather(entity_table, idx2d)
    ent = ent_flat.reshape(_BATCH, _FIELDS, _D)
    z_R, z_C = _tc_proj(R, W_p.T, C_R.astype(R.dtype), W_p_c.T)
    return ent, z_R, z_C


# speedup vs baseline: 1.5744x; 1.0087x over previous
import functools

import jax
import jax.numpy as jnp
from jax import lax
from jax.experimental import pallas as pl
from jax.experimental.pallas import tpu as pltpu
from jax.experimental.pallas import tpu_sc as plsc

_BATCH = 16384
_FIELDS = 26
_D = 32
_BF = _BATCH * _FIELDS

_NC = 2
_NS = 16
_NW = _NC * _NS
_PER_W = _BF // _NW
_ILEN = 128
_IDX_ROWS = _PER_W // _ILEN
_SUB = 4
_CHUNK = _SUB * _ILEN
_OUTER = _PER_W // _CHUNK


def _gather_body(table_hbm, idx_hbm, out_hbm, idx_v, rows0, rows1, g0, g1):
    wid = lax.axis_index("s") * _NC + lax.axis_index("c")
    base = wid * _PER_W
    pltpu.sync_copy(idx_hbm.at[pl.ds(wid * _IDX_ROWS, _IDX_ROWS)], idx_v)

    def fire(t, rows_ref, sem):
        for b in range(_SUB):
            pltpu.async_copy(
                table_hbm.at[idx_v.at[t * _SUB + b]],
                rows_ref.at[pl.ds(b * _ILEN, _ILEN)],
                sem,
            )

    def drain(rows_ref, sem):
        pltpu.make_async_copy(
            table_hbm.at[pl.ds(0, _CHUNK)], rows_ref, sem
        ).wait()

    def writeback(t, rows_ref):
        pltpu.sync_copy(rows_ref, out_hbm.at[pl.ds(base + t * _CHUNK, _CHUNK)])

    fire(0, rows0, g0)
    fire(1, rows1, g1)

    def body(i, carry):
        t = 2 * i
        drain(rows0, g0)
        writeback(t, rows0)
        fire(t + 2, rows0, g0)
        drain(rows1, g1)
        writeback(t + 1, rows1)
        fire(t + 3, rows1, g1)
        return carry

    lax.fori_loop(0, _OUTER // 2 - 1, body, 0)

    drain(rows0, g0)
    writeback(_OUTER - 2, rows0)
    drain(rows1, g1)
    writeback(_OUTER - 1, rows1)


@functools.partial(
    pl.kernel,
    out_type=jax.ShapeDtypeStruct((_BF, _D), jnp.float32),
    mesh=plsc.VectorSubcoreMesh(core_axis_name="c", subcore_axis_name="s"),
    compiler_params=pltpu.CompilerParams(use_tc_tiling_on_sc=False),
    scratch_types=[
        pltpu.VMEM((_IDX_ROWS, _ILEN), jnp.int32),
        pltpu.VMEM((_CHUNK, _D), jnp.float32),
        pltpu.VMEM((_CHUNK, _D), jnp.float32),
        pltpu.SemaphoreType.DMA,
        pltpu.SemaphoreType.DMA,
    ],
)
def _sc_gather(table_hbm, idx_hbm, out_hbm, idx_v, rows0, rows1, g0, g1):
    _gather_body(table_hbm, idx_hbm, out_hbm, idx_v, rows0, rows1, g0, g1)


def _proj_body(r_ref, wpt_ref, cr_ref, wpct_ref, zr_ref, zc_ref):
    zr_ref[...] = jnp.dot(r_ref[...], wpt_ref[...],
                          preferred_element_type=jnp.float32)
    zc_ref[...] = jnp.dot(cr_ref[...], wpct_ref[...],
                          preferred_element_type=jnp.float32)


_tc_proj = pl.pallas_call(
    _proj_body,
    out_shape=[
        jax.ShapeDtypeStruct((500, 128), jnp.float32),
        jax.ShapeDtypeStruct((500, 128), jnp.float32),
    ],
)


def kernel(indices, R, C_R, entity_table, W_p, W_p_c):
    idx2d = indices.reshape(_BF // _ILEN, _ILEN).astype(jnp.int32)
    ent_flat = _sc_gather(entity_table, idx2d)
    ent = ent_flat.reshape(_BATCH, _FIELDS, _D)
    z_R, z_C = _tc_proj(R, W_p.T, C_R.astype(R.dtype), W_p_c.T)
    return ent, z_R, z_C


# speedup vs baseline: 1.6713x; 1.0616x over previous
import functools

import jax
import jax.numpy as jnp
from jax import lax
from jax.experimental import pallas as pl
from jax.experimental.pallas import tpu as pltpu
from jax.experimental.pallas import tpu_sc as plsc

_BATCH = 16384
_FIELDS = 26
_D = 32
_BF = _BATCH * _FIELDS

_NC = 2
_NS = 16
_NW = _NC * _NS
_PER_W = _BF // _NW
_ILEN = 128
_IDX_ROWS = _PER_W // _ILEN
_SUB = 4
_CHUNK = _SUB * _ILEN
_OUTER = _PER_W // _CHUNK


def _gather_body(table_hbm, idx_hbm, out_hbm, idx_v, rows0, rows1, g0, g1):
    wid = lax.axis_index("s") * _NC + lax.axis_index("c")
    base = wid * _PER_W
    pltpu.sync_copy(idx_hbm.at[pl.ds(wid * _IDX_ROWS, _IDX_ROWS)], idx_v)

    def fire(t, rows_ref, sem):
        for b in range(_SUB):
            pltpu.async_copy(
                table_hbm.at[idx_v.at[t * _SUB + b]],
                rows_ref.at[pl.ds(b * _ILEN, _ILEN)],
                sem,
            )

    def drain(rows_ref, sem):
        pltpu.make_async_copy(
            table_hbm.at[pl.ds(0, _CHUNK)], rows_ref, sem
        ).wait()

    def writeback(t, rows_ref):
        pltpu.sync_copy(rows_ref, out_hbm.at[pl.ds(base + t * _CHUNK, _CHUNK)])

    fire(0, rows0, g0)
    fire(1, rows1, g1)

    def body(i, carry):
        t = 2 * i
        drain(rows0, g0)
        writeback(t, rows0)
        fire(t + 2, rows0, g0)
        drain(rows1, g1)
        writeback(t + 1, rows1)
        fire(t + 3, rows1, g1)
        return carry

    lax.fori_loop(0, _OUTER // 2 - 1, body, 0)

    drain(rows0, g0)
    writeback(_OUTER - 2, rows0)
    drain(rows1, g1)
    writeback(_OUTER - 1, rows1)


@functools.partial(
    pl.kernel,
    out_type=jax.ShapeDtypeStruct((_BF, _D), jnp.float32),
    mesh=plsc.VectorSubcoreMesh(core_axis_name="c", subcore_axis_name="s"),
    compiler_params=pltpu.CompilerParams(use_tc_tiling_on_sc=False),
    scratch_types=[
        pltpu.VMEM((_IDX_ROWS, _ILEN), jnp.int32),
        pltpu.VMEM((_CHUNK, _D), jnp.float32),
        pltpu.VMEM((_CHUNK, _D), jnp.float32),
        pltpu.SemaphoreType.DMA,
        pltpu.SemaphoreType.DMA,
    ],
)
def _sc_gather(table_hbm, idx_hbm, out_hbm, idx_v, rows0, rows1, g0, g1):
    _gather_body(table_hbm, idx_hbm, out_hbm, idx_v, rows0, rows1, g0, g1)


def _proj_body(r_ref, wpt_ref, cr_ref, wpct_ref, zr_ref, zc_ref):
    zr_ref[...] = jnp.dot(r_ref[...], wpt_ref[...],
                          preferred_element_type=jnp.float32)
    zc_ref[...] = jnp.dot(cr_ref[...], wpct_ref[...],
                          preferred_element_type=jnp.float32)


_tc_proj = pl.pallas_call(
    _proj_body,
    out_shape=[
        jax.ShapeDtypeStruct((500, 128), jnp.float32),
        jax.ShapeDtypeStruct((500, 128), jnp.float32),
    ],
)


def kernel(indices, R, C_R, entity_table, W_p, W_p_c):
    idx2d = indices.T.reshape(_BF // _ILEN, _ILEN).astype(jnp.int32)
    ent_flat = _sc_gather(entity_table, idx2d)
    ent = ent_flat.reshape(_FIELDS, _BATCH, _D).transpose(1, 0, 2)
    z_R, z_C = _tc_proj(R, W_p.T, C_R.astype(R.dtype), W_p_c.T)
    return ent, z_R, z_C
